# Initial kernel scaffold; baseline (speedup 1.0000x reference)
import jax
import jax.numpy as jnp
from jax.experimental import pallas as pl


def kernel(x_ids, table):
    raise NotImplementedError("write your pallas kernel here")


# speedup vs baseline: 1.1137x; 1.1137x over previous
import functools

import jax
import jax.numpy as jnp
from jax import lax
from jax.experimental import pallas as pl
from jax.experimental.pallas import tpu as pltpu
from jax.experimental.pallas import tpu_sc as plsc

EMB = 32
CHUNK = 128
NBUF = 8


@functools.partial(jax.jit, static_argnames=())
def _embed_flat(idx2d, table):
    nrows, chunk = idx2d.shape
    info = plsc.get_sparse_core_info()
    ncores, nsub = info.num_cores, info.num_subcores
    nw = ncores * nsub
    rpw = nrows // nw

    mesh = plsc.VectorSubcoreMesh(core_axis_name="c", subcore_axis_name="s")

    @functools.partial(
        pl.kernel,
        out_type=jax.ShapeDtypeStruct((nrows * chunk, EMB), jnp.float32),
        mesh=mesh,
        scratch_types=(
            [pltpu.VMEM((rpw, chunk), jnp.int32),
             pltpu.VMEM((NBUF, chunk, EMB), jnp.float32)]
            + [pltpu.SemaphoreType.DMA] * NBUF
        ),
        compiler_params=pltpu.CompilerParams(use_tc_tiling_on_sc=False),
    )
    def body(idx_hbm, table_hbm, out_hbm, idx_v, rows_v, *gsems):
        wid = lax.axis_index("s") * ncores + lax.axis_index("c")
        r0 = wid * rpw
        pltpu.sync_copy(idx_hbm.at[pl.ds(r0, rpw), :], idx_v)

        for b in range(NBUF):
            pltpu.async_copy(table_hbm.at[idx_v.at[b]], rows_v.at[b], gsems[b])

        @pl.loop(0, rpw, step=NBUF)
        def _(g):
            for b in range(NBUF):
                r = g + b
                pltpu.make_async_copy(
                    table_hbm.at[idx_v.at[r]], rows_v.at[b], gsems[b]
                ).wait()
                pltpu.sync_copy(
                    rows_v.at[b], out_hbm.at[pl.ds((r0 + r) * chunk, chunk), :]
                )
                nxt = r + NBUF

                @pl.when(nxt < rpw)
                def _():
                    pltpu.async_copy(
                        table_hbm.at[idx_v.at[nxt]], rows_v.at[b], gsems[b]
                    )

    return body(idx2d, table)


def kernel(x_ids, table):
    batch, seq = x_ids.shape
    flat = x_ids.reshape(-1).astype(jnp.int32)
    idx2d = flat.reshape(-1, CHUNK)
    out = _embed_flat(idx2d, table)
    return out.reshape(batch, seq, EMB)
